# baseline (speedup 1.0000x reference)
import jax
import jax.numpy as jnp
from jax.experimental import pallas as pl
from jax.experimental.pallas import tpu as pltpu

_TOPK = 10
_S = 8
_G = 16
_KEEP = 3
_NB = 512


def _merge_sorted(a, b, keep):
    out = []
    for j in range(keep):
        terms = []
        if j < len(a):
            terms.append(a[j])
        if j < len(b):
            terms.append(b[j])
        for i in range(j):
            if i < len(a) and (j - 1 - i) < len(b):
                terms.append(jnp.minimum(a[i], b[j - 1 - i]))
        r = terms[0]
        for x in terms[1:]:
            r = jnp.maximum(r, x)
        out.append(r)
    return out


def _top3_of8(vs):
    pairs = []
    for i in range(4):
        hi = jnp.maximum(vs[2 * i], vs[2 * i + 1])
        lo = jnp.minimum(vs[2 * i], vs[2 * i + 1])
        pairs.append([hi, lo])
    m0 = _merge_sorted(pairs[0], pairs[1], 3)
    m1 = _merge_sorted(pairs[2], pairs[3], 3)
    return _merge_sorted(m0, m1, 3)


def _insertion_topk(read, nchunks, s, nb, unroll):
    def _insert(i, t):
        v = read(i)
        t = list(t)
        for j in range(_TOPK):
            hi = jnp.maximum(t[j], v)
            v = jnp.minimum(t[j], v)
            t[j] = hi
        return tuple(t)

    t0 = tuple(
        jnp.full((s, nb), -jnp.inf, dtype=jnp.float32) for _ in range(_TOPK)
    )
    return jax.lax.fori_loop(0, nchunks, _insert, t0, unroll=unroll)


def _extract_threshold(t, nb):
    vals = jnp.concatenate(list(t), axis=0)
    need = jnp.full((1, nb), float(_TOPK), dtype=jnp.float32)
    th = jnp.full((1, nb), -jnp.inf, dtype=jnp.float32)
    for _ in range(_TOPK):
        m = jnp.max(vals, axis=0, keepdims=True)
        ge = vals >= m
        c = jnp.sum(ge.astype(jnp.float32), axis=0, keepdims=True)
        th = jnp.where(need > 0.0, m, th)
        need = need - c
        vals = jnp.where(ge, -jnp.inf, vals)
    return th


def _assemble_body(ncols, aff_ref, feat_ref, out_ref, cand_ref):
    p = aff_ref.shape[1]
    nb = aff_ref.shape[2]
    full_groups = p // (_S * _G)
    tail_chunks = (p - full_groups * _S * _G) // _S

    def _prefilter(g, carry):
        base = g * _S * _G
        vs = [aff_ref[0, pl.ds(base + j * _S, _S), :] for j in range(8)]
        t0 = _top3_of8(vs)
        vs = [aff_ref[0, pl.ds(base + (8 + j) * _S, _S), :] for j in range(8)]
        t1 = _top3_of8(vs)
        top = _merge_sorted(t0, t1, _KEEP)
        for j in range(_KEEP):
            cand_ref[pl.ds((g * _KEEP + j) * _S, _S), :] = top[j]
        return carry

    jax.lax.fori_loop(0, full_groups, _prefilter, 0, unroll=2)
    if tail_chunks:
        base = full_groups * _S * _G
        vs = [
            aff_ref[0, pl.ds(base + j * _S, _S), :] for j in range(tail_chunks)
        ]
        top = _top3_of8(vs)
        for j in range(_KEEP):
            cand_ref[pl.ds(((full_groups * _KEEP) + j) * _S, _S), :] = top[j]

    ncand = (full_groups + 1) * _KEEP

    t = _insertion_topk(
        lambda i: cand_ref[pl.ds(i * _S, _S), :], ncand, _S, nb, 8
    )
    th1 = _extract_threshold(t, nb)
    mx = jnp.max(t[0], axis=0, keepdims=True)

    a = aff_ref[0]
    ex = jnp.exp(a - mx)
    e1 = jnp.where(a >= th1, ex, 0.0)
    cnt_gt = jnp.sum((a > th1).astype(jnp.float32), axis=0, keepdims=True)
    base_col = pl.program_id(1) * nb
    colid = jax.lax.broadcasted_iota(jnp.int32, (1, nb), 1) + base_col
    valid = colid < ncols
    ok = jnp.all(jnp.where(valid, cnt_gt, 0.0) < float(_TOPK))

    def _fallback():
        tf = _insertion_topk(
            lambda i: aff_ref[0, pl.ds(i * _S, _S), :], p // _S, _S, nb, 8
        )
        th2 = _extract_threshold(tf, nb)
        return jnp.where(a >= th2, ex, 0.0)

    e = jax.lax.cond(ok, lambda: e1, _fallback)
    ssum = jnp.sum(e, axis=0, keepdims=True)
    f = feat_ref[0]
    acc = jax.lax.dot_general(
        f, e, (((1,), (0,)), ((), ())), preferred_element_type=jnp.float32
    )
    out_ref[0] = acc * (1.0 / ssum)


def kernel(cur_prev_aff, feat):
    import functools

    B, P, N = cur_prev_aff.shape
    C = feat.shape[1]
    NB = _NB
    grid = (B, pl.cdiv(N, NB))
    ncand_chunks = (P // (_S * _G) + 1) * _KEEP
    return pl.pallas_call(
        functools.partial(_assemble_body, N),
        grid=grid,
        in_specs=[
            pl.BlockSpec((1, P, NB), lambda b, n: (b, 0, n)),
            pl.BlockSpec((1, C, P), lambda b, n: (b, 0, 0)),
        ],
        out_specs=pl.BlockSpec((1, C, NB), lambda b, n: (b, 0, n)),
        out_shape=jax.ShapeDtypeStruct((B, C, N), jnp.float32),
        scratch_shapes=[pltpu.VMEM((ncand_chunks * _S, NB), jnp.float32)],
    )(cur_prev_aff, feat)


# speedup vs baseline: 1.2387x; 1.2387x over previous
import jax
import jax.numpy as jnp
from jax.experimental import pallas as pl
from jax.experimental.pallas import tpu as pltpu

_TOPK = 10
_S = 8
_G = 16
_KEEP = 3
_NB = 512


def _merge_sorted(a, b, keep):
    out = []
    for j in range(keep):
        terms = []
        if j < len(a):
            terms.append(a[j])
        if j < len(b):
            terms.append(b[j])
        for i in range(j):
            if i < len(a) and (j - 1 - i) < len(b):
                terms.append(jnp.minimum(a[i], b[j - 1 - i]))
        r = terms[0]
        for x in terms[1:]:
            r = jnp.maximum(r, x)
        out.append(r)
    return out


def _top3_of8(vs):
    pairs = []
    for i in range(4):
        hi = jnp.maximum(vs[2 * i], vs[2 * i + 1])
        lo = jnp.minimum(vs[2 * i], vs[2 * i + 1])
        pairs.append([hi, lo])
    m0 = _merge_sorted(pairs[0], pairs[1], 3)
    m1 = _merge_sorted(pairs[2], pairs[3], 3)
    return _merge_sorted(m0, m1, 3)


def _insertion_topk(read, nchunks, s, nb, unroll):
    def _insert(i, t):
        v = read(i)
        t = list(t)
        for j in range(_TOPK):
            hi = jnp.maximum(t[j], v)
            v = jnp.minimum(t[j], v)
            t[j] = hi
        return tuple(t)

    t0 = tuple(
        jnp.full((s, nb), -jnp.inf, dtype=jnp.float32) for _ in range(_TOPK)
    )
    return jax.lax.fori_loop(0, nchunks, _insert, t0, unroll=unroll)


def _extract_threshold(t, nb):
    lists = list(t)
    for shift in (4, 2, 1):
        rot = [pltpu.roll(x, shift, 0) for x in lists]
        lists = _merge_sorted(lists, rot, _TOPK)
    return lists[_TOPK - 1][0:1, :], lists[0][0:1, :]


def _assemble_body(ncols, aff_ref, feat_ref, out_ref, cand_ref):
    p = aff_ref.shape[1]
    nb = aff_ref.shape[2]
    full_groups = p // (_S * _G)
    tail_chunks = (p - full_groups * _S * _G) // _S

    def _prefilter(g, carry):
        base = g * _S * _G
        vs = [aff_ref[0, pl.ds(base + j * _S, _S), :] for j in range(8)]
        t0 = _top3_of8(vs)
        vs = [aff_ref[0, pl.ds(base + (8 + j) * _S, _S), :] for j in range(8)]
        t1 = _top3_of8(vs)
        top = _merge_sorted(t0, t1, _KEEP)
        for j in range(_KEEP):
            cand_ref[pl.ds((g * _KEEP + j) * _S, _S), :] = top[j]
        return carry

    jax.lax.fori_loop(0, full_groups, _prefilter, 0, unroll=2)
    if tail_chunks:
        base = full_groups * _S * _G
        vs = [
            aff_ref[0, pl.ds(base + j * _S, _S), :] for j in range(tail_chunks)
        ]
        top = _top3_of8(vs)
        for j in range(_KEEP):
            cand_ref[pl.ds(((full_groups * _KEEP) + j) * _S, _S), :] = top[j]

    ncand = (full_groups + 1) * _KEEP

    t = _insertion_topk(
        lambda i: cand_ref[pl.ds(i * _S, _S), :], ncand, _S, nb, 8
    )
    th1, mx = _extract_threshold(t, nb)

    a = aff_ref[0]
    cnt_gt = jnp.sum((a > th1).astype(jnp.float32), axis=0, keepdims=True)
    base_col = pl.program_id(1) * nb
    colid = jax.lax.broadcasted_iota(jnp.int32, (1, nb), 1) + base_col
    valid = colid < ncols
    ok = jnp.all(jnp.where(valid, cnt_gt, 0.0) < float(_TOPK))

    def _fallback():
        tf = _insertion_topk(
            lambda i: aff_ref[0, pl.ds(i * _S, _S), :], p // _S, _S, nb, 8
        )
        return _extract_threshold(tf, nb)[0]

    th = jax.lax.cond(ok, lambda: th1, _fallback)

    e = jnp.where(a >= th, jnp.exp(a - mx), 0.0)
    ssum = jnp.sum(e, axis=0, keepdims=True)
    f = feat_ref[0]
    acc = jax.lax.dot_general(
        f, e, (((1,), (0,)), ((), ())), preferred_element_type=jnp.float32
    )
    out_ref[0] = acc * (1.0 / ssum)


def kernel(cur_prev_aff, feat):
    import functools

    B, P, N = cur_prev_aff.shape
    C = feat.shape[1]
    NB = _NB
    grid = (B, pl.cdiv(N, NB))
    ncand_chunks = (P // (_S * _G) + 1) * _KEEP
    return pl.pallas_call(
        functools.partial(_assemble_body, N),
        grid=grid,
        in_specs=[
            pl.BlockSpec((1, P, NB), lambda b, n: (b, 0, n)),
            pl.BlockSpec((1, C, P), lambda b, n: (b, 0, 0)),
        ],
        out_specs=pl.BlockSpec((1, C, NB), lambda b, n: (b, 0, n)),
        out_shape=jax.ShapeDtypeStruct((B, C, N), jnp.float32),
        scratch_shapes=[pltpu.VMEM((ncand_chunks * _S, NB), jnp.float32)],
    )(cur_prev_aff, feat)
